# baseline (speedup 1.0000x reference)
import numpy as np
import jax
import jax.numpy as jnp
from jax.experimental import pallas as pl
from jax.experimental.pallas import tpu as pltpu

_B, _T, _C, _H, _W = 4, 4, 8, 224, 224
_R = _B * _T
_N = _H * _W
_K = _N // 2
_DENOM = float(_R * _K * _C)


def _np_threefry2x32(k0, k1, x0, x1):
    def rotl(x, d):
        return ((x << np.uint32(d)) | (x >> np.uint32(32 - d))).astype(np.uint32)

    ks0, ks1 = np.uint32(k0), np.uint32(k1)
    ks2 = np.uint32(ks0 ^ ks1 ^ np.uint32(0x1BD11BDA))
    ks = [ks0, ks1, ks2]
    rotations = [(13, 15, 26, 6), (17, 29, 16, 24)]
    x0 = (x0 + ks0).astype(np.uint32)
    x1 = (x1 + ks1).astype(np.uint32)
    for i in range(5):
        for r in rotations[i % 2]:
            x0 = (x0 + x1).astype(np.uint32)
            x1 = rotl(x1, r)
            x1 = (x1 ^ x0).astype(np.uint32)
        x0 = (x0 + ks[(i + 1) % 3]).astype(np.uint32)
        x1 = (x1 + ks[(i + 2) % 3] + np.uint32(i + 1)).astype(np.uint32)
    return x0, x1


def _np_uniform_table(seed, size):
    k0 = np.uint32(np.uint64(seed) >> np.uint64(32))
    k1 = np.uint32(np.uint64(seed) & np.uint64(0xFFFFFFFF))
    lo = np.arange(size, dtype=np.uint32)
    hi = np.zeros(size, dtype=np.uint32)
    o0, o1 = _np_threefry2x32(k0, k1, hi, lo)
    bits = o0 ^ o1
    fb = (bits >> np.uint32(9)) | np.uint32(0x3F800000)
    floats = fb.view(np.float32) - np.float32(1.0)
    tiny = np.float32(np.finfo(np.float32).tiny)
    return np.maximum(tiny, floats * (np.float32(1.0) - tiny) + tiny)


_U = _np_uniform_table(42, _R * _N).reshape(_R, _H, _W)

_INT_MIN = np.int32(-2147483648)


def _mae_body(t_ref, p_ref, g_ref, o_ref, key_ref, d_ref):
    r = pl.program_id(0)
    t = t_ref[0]
    p = p_ref[0]
    g = g_ref[0]

    norm = jnp.sqrt(jnp.sum(t * t, axis=0)) + 0.5
    gumb = -jnp.log(-jnp.log(g))
    score = jnp.log(norm) + gumb
    d = jnp.sum(jnp.abs(p - t), axis=0)

    u = jax.lax.bitcast_convert_type(score, jnp.int32)
    key_ref[r] = jnp.where(u >= 0, u, _INT_MIN - u)
    d_ref[r] = d

    @pl.when(r == _R - 1)
    def _phase_b():
        key = key_ref[...]
        dd = d_ref[...]

        hi16 = (key >> 16).astype(jnp.int16)
        half = _H // 2

        def cntmask(m):
            c1 = jnp.sum(m[:, :half].astype(jnp.int16), axis=(1, 2),
                         keepdims=True).astype(jnp.int32)
            c2 = jnp.sum(m[:, half:].astype(jnp.int16), axis=(1, 2),
                         keepdims=True).astype(jnp.int32)
            return c1 + c2

        lo0 = jnp.full((_R, 1, 1), -32768, jnp.int32)
        hi0 = jnp.full((_R, 1, 1), 32768, jnp.int32)

        def body1(_, lohi):
            lo, hi = lohi
            mid = (lo + hi) >> 1
            pred = cntmask(hi16 >= mid.astype(jnp.int16)) >= _K
            return jnp.where(pred, mid, lo), jnp.where(pred, hi, mid)

        p_hi, _ = jax.lax.fori_loop(0, 16, body1, (lo0, hi0))
        p16 = p_hi.astype(jnp.int16)
        c_gt_hi = cntmask(hi16 > p16)
        need2 = _K - c_gt_hi

        lo16m = ((key & 0xFFFF) - 32768).astype(jnp.int16)
        cand = jnp.where(hi16 == p16, lo16m, jnp.int16(-32768))

        def body2(_, lohi):
            lo, hi = lohi
            mid = (lo + hi) >> 1
            pred = cntmask(cand >= mid.astype(jnp.int16)) >= need2
            return jnp.where(pred, mid, lo), jnp.where(pred, hi, mid)

        lo2, _ = jax.lax.fori_loop(0, 16, body2, (lo0, hi0))
        tau = (p_hi << 16) + (lo2 + 32768)

        mask_gt = key > tau
        mask_eq = key == tau
        count_gt = jnp.sum(mask_gt.astype(jnp.float32), axis=(1, 2),
                           keepdims=True)
        count_eq = jnp.sum(mask_eq.astype(jnp.float32), axis=(1, 2),
                           keepdims=True)
        sum_gt = jnp.sum(jnp.where(mask_gt, dd, 0.0), axis=(1, 2),
                         keepdims=True)
        sum_eq = jnp.sum(jnp.where(mask_eq, dd, 0.0), axis=(1, 2),
                         keepdims=True)
        need = jnp.float32(_K) - count_gt
        total = jnp.sum(sum_gt + need * sum_eq / count_eq)
        o_ref[0, 0] = total * (1.0 / _DENOM)


def kernel(out_preds, out_targets, tl, tv, x_rep, in_x, in_l, in_v, in_n):
    t = out_targets.reshape(_R, _C, _H, _W)
    p = out_preds.reshape(_R, _C, _H, _W)
    g = jnp.asarray(_U)
    out = pl.pallas_call(
        _mae_body,
        grid=(_R,),
        in_specs=[
            pl.BlockSpec((1, _C, _H, _W), lambda r: (r, 0, 0, 0)),
            pl.BlockSpec((1, _C, _H, _W), lambda r: (r, 0, 0, 0)),
            pl.BlockSpec((1, _H, _W), lambda r: (r, 0, 0)),
        ],
        out_specs=pl.BlockSpec((1, 1), lambda r: (0, 0), memory_space=pltpu.SMEM),
        out_shape=jax.ShapeDtypeStruct((1, 1), jnp.float32),
        scratch_shapes=[
            pltpu.VMEM((_R, _H, _W), jnp.int32),
            pltpu.VMEM((_R, _H, _W), jnp.float32),
        ],
    )(t, p, g)
    return out[0, 0]


# speedup vs baseline: 1.3889x; 1.3889x over previous
import numpy as np
import jax
import jax.numpy as jnp
from jax.experimental import pallas as pl
from jax.experimental.pallas import tpu as pltpu

_B, _T, _C, _H, _W = 4, 4, 8, 224, 224
_R = _B * _T
_N = _H * _W
_K = _N // 2
_DENOM = float(_R * _K * _C)


def _np_threefry2x32(k0, k1, x0, x1):
    def rotl(x, d):
        return ((x << np.uint32(d)) | (x >> np.uint32(32 - d))).astype(np.uint32)

    ks0, ks1 = np.uint32(k0), np.uint32(k1)
    ks2 = np.uint32(ks0 ^ ks1 ^ np.uint32(0x1BD11BDA))
    ks = [ks0, ks1, ks2]
    rotations = [(13, 15, 26, 6), (17, 29, 16, 24)]
    x0 = (x0 + ks0).astype(np.uint32)
    x1 = (x1 + ks1).astype(np.uint32)
    for i in range(5):
        for r in rotations[i % 2]:
            x0 = (x0 + x1).astype(np.uint32)
            x1 = rotl(x1, r)
            x1 = (x1 ^ x0).astype(np.uint32)
        x0 = (x0 + ks[(i + 1) % 3]).astype(np.uint32)
        x1 = (x1 + ks[(i + 2) % 3] + np.uint32(i + 1)).astype(np.uint32)
    return x0, x1


def _np_uniform_table(seed, size):
    k0 = np.uint32(np.uint64(seed) >> np.uint64(32))
    k1 = np.uint32(np.uint64(seed) & np.uint64(0xFFFFFFFF))
    lo = np.arange(size, dtype=np.uint32)
    hi = np.zeros(size, dtype=np.uint32)
    o0, o1 = _np_threefry2x32(k0, k1, hi, lo)
    bits = o0 ^ o1
    fb = (bits >> np.uint32(9)) | np.uint32(0x3F800000)
    floats = fb.view(np.float32) - np.float32(1.0)
    tiny = np.float32(np.finfo(np.float32).tiny)
    return np.maximum(tiny, floats * (np.float32(1.0) - tiny) + tiny)


_U = _np_uniform_table(42, _R * _N).reshape(_R, _H, _W)

_INT_MIN = np.int32(-2147483648)


_INT_MAX = np.int32(2147483647)

_G1 = 8
_SCHED = {8: 5, 9: 5, 10: 5, 11: 5, 12: 4, 13: 4, 14: 4}


def _search_body(key):
    def body(_, lohi):
        lo, hi = lohi
        mid = (lo >> 1) + (hi >> 1) + (lo & hi & 1)
        cnt = jnp.sum((key >= mid).astype(jnp.int32), axis=(1, 2),
                      keepdims=True)
        pred = cnt >= _K
        return jnp.where(pred, mid, lo), jnp.where(pred, hi, mid)

    return body


def _mae_body(t_ref, p_ref, g_ref, o_ref, key_ref, d_ref, lo_ref, hi_ref):
    r = pl.program_id(0)
    t = t_ref[0]
    p = p_ref[0]
    g = g_ref[0]

    norm = jnp.sqrt(jnp.sum(t * t, axis=0)) + 0.5
    gumb = -jnp.log(-jnp.log(g))
    score = jnp.log(norm) + gumb
    d = jnp.sum(jnp.abs(p - t), axis=0)

    u = jax.lax.bitcast_convert_type(score, jnp.int32)
    key_ref[r] = jnp.where(u >= 0, u, _INT_MIN - u)
    d_ref[r] = d

    @pl.when(r == min(_SCHED))
    def _init_g1():
        lo_ref[...] = jnp.full((_G1, 1, 1), _INT_MIN, jnp.int32)
        hi_ref[...] = jnp.full((_G1, 1, 1), _INT_MAX, jnp.int32)

    for _step, _n in _SCHED.items():
        @pl.when(r == _step)
        def _advance_g1(_n=_n):
            k1 = key_ref[pl.ds(0, _G1)]
            lo, hi = jax.lax.fori_loop(
                0, _n, _search_body(k1), (lo_ref[...], hi_ref[...]))
            lo_ref[...] = lo
            hi_ref[...] = hi

    @pl.when(r == _R - 1)
    def _phase_b():
        k2 = key_ref[pl.ds(_G1, _R - _G1)]
        lo0 = jnp.full((_R - _G1, 1, 1), _INT_MIN, jnp.int32)
        hi0 = jnp.full((_R - _G1, 1, 1), _INT_MAX, jnp.int32)
        tau2, _ = jax.lax.fori_loop(0, 32, _search_body(k2), (lo0, hi0))

        key = key_ref[...]
        dd = d_ref[...]
        tau = jnp.concatenate([lo_ref[...], tau2], axis=0)

        mask_ge = key >= tau
        count_ge = jnp.sum(mask_ge.astype(jnp.int32), axis=(1, 2),
                           keepdims=True)
        sum_ge = jnp.sum(jnp.where(mask_ge, dd, 0.0), axis=(1, 2),
                         keepdims=True)
        exact = jnp.all(count_ge == _K)

        @pl.when(exact)
        def _no_ties():
            o_ref[0, 0] = jnp.sum(sum_ge) * (1.0 / _DENOM)

        @pl.when(jnp.logical_not(exact))
        def _ties():
            mask_eq = key == tau
            count_eq = jnp.sum(mask_eq.astype(jnp.float32), axis=(1, 2),
                               keepdims=True)
            sum_eq = jnp.sum(jnp.where(mask_eq, dd, 0.0), axis=(1, 2),
                             keepdims=True)
            count_gt = count_ge.astype(jnp.float32) - count_eq
            sum_gt = sum_ge - sum_eq
            need = jnp.float32(_K) - count_gt
            total = jnp.sum(sum_gt + need * sum_eq / count_eq)
            o_ref[0, 0] = total * (1.0 / _DENOM)


def kernel(out_preds, out_targets, tl, tv, x_rep, in_x, in_l, in_v, in_n):
    t = out_targets.reshape(_R, _C, _H, _W)
    p = out_preds.reshape(_R, _C, _H, _W)
    g = jnp.asarray(_U)
    out = pl.pallas_call(
        _mae_body,
        grid=(_R,),
        in_specs=[
            pl.BlockSpec((1, _C, _H, _W), lambda r: (r, 0, 0, 0)),
            pl.BlockSpec((1, _C, _H, _W), lambda r: (r, 0, 0, 0)),
            pl.BlockSpec((1, _H, _W), lambda r: (r, 0, 0)),
        ],
        out_specs=pl.BlockSpec((1, 1), lambda r: (0, 0), memory_space=pltpu.SMEM),
        out_shape=jax.ShapeDtypeStruct((1, 1), jnp.float32),
        scratch_shapes=[
            pltpu.VMEM((_R, _H, _W), jnp.int32),
            pltpu.VMEM((_R, _H, _W), jnp.float32),
            pltpu.VMEM((_G1, 1, 1), jnp.int32),
            pltpu.VMEM((_G1, 1, 1), jnp.int32),
        ],
    )(t, p, g)
    return out[0, 0]


# speedup vs baseline: 1.4104x; 1.0155x over previous
import numpy as np
import jax
import jax.numpy as jnp
from jax.experimental import pallas as pl
from jax.experimental.pallas import tpu as pltpu

_B, _T, _C, _H, _W = 4, 4, 8, 224, 224
_R = _B * _T
_N = _H * _W
_K = _N // 2
_DENOM = float(_R * _K * _C)


def _np_threefry2x32(k0, k1, x0, x1):
    def rotl(x, d):
        return ((x << np.uint32(d)) | (x >> np.uint32(32 - d))).astype(np.uint32)

    ks0, ks1 = np.uint32(k0), np.uint32(k1)
    ks2 = np.uint32(ks0 ^ ks1 ^ np.uint32(0x1BD11BDA))
    ks = [ks0, ks1, ks2]
    rotations = [(13, 15, 26, 6), (17, 29, 16, 24)]
    x0 = (x0 + ks0).astype(np.uint32)
    x1 = (x1 + ks1).astype(np.uint32)
    for i in range(5):
        for r in rotations[i % 2]:
            x0 = (x0 + x1).astype(np.uint32)
            x1 = rotl(x1, r)
            x1 = (x1 ^ x0).astype(np.uint32)
        x0 = (x0 + ks[(i + 1) % 3]).astype(np.uint32)
        x1 = (x1 + ks[(i + 2) % 3] + np.uint32(i + 1)).astype(np.uint32)
    return x0, x1


def _np_uniform_table(seed, size):
    k0 = np.uint32(np.uint64(seed) >> np.uint64(32))
    k1 = np.uint32(np.uint64(seed) & np.uint64(0xFFFFFFFF))
    lo = np.arange(size, dtype=np.uint32)
    hi = np.zeros(size, dtype=np.uint32)
    o0, o1 = _np_threefry2x32(k0, k1, hi, lo)
    bits = o0 ^ o1
    fb = (bits >> np.uint32(9)) | np.uint32(0x3F800000)
    floats = fb.view(np.float32) - np.float32(1.0)
    tiny = np.float32(np.finfo(np.float32).tiny)
    return np.maximum(tiny, floats * (np.float32(1.0) - tiny) + tiny)


_E = (np.float32(-1.0) / np.log(_np_uniform_table(42, _R * _N))).reshape(
    _R, _H, _W)


_INT_MAX = np.int32(2147483647)

_G1 = 8
_SCHED = {8: 5, 9: 5, 10: 5, 11: 5, 12: 4, 13: 4, 14: 4}


def _search_body(key):
    def body(_, lohi):
        lo, hi = lohi
        mid = (lo >> 1) + (hi >> 1) + (lo & hi & 1)
        cnt = jnp.sum((key >= mid).astype(jnp.int32), axis=(1, 2),
                      keepdims=True)
        pred = cnt >= _K
        return jnp.where(pred, mid, lo), jnp.where(pred, hi, mid)

    return body


def _mae_body(t_ref, p_ref, g_ref, o_ref, key_ref, d_ref, lo_ref, hi_ref):
    r = pl.program_id(0)
    t = t_ref[0]
    p = p_ref[0]
    g = g_ref[0]

    v = (jnp.sqrt(jnp.sum(t * t, axis=0)) + 0.5) * g
    d = jnp.sum(jnp.abs(p - t), axis=0)

    key_ref[r] = jax.lax.bitcast_convert_type(v, jnp.int32)
    d_ref[r] = d

    @pl.when(r == min(_SCHED))
    def _init_g1():
        lo_ref[...] = jnp.zeros((_G1, 1, 1), jnp.int32)
        hi_ref[...] = jnp.full((_G1, 1, 1), _INT_MAX, jnp.int32)

    for _step, _n in _SCHED.items():
        @pl.when(r == _step)
        def _advance_g1(_n=_n):
            k1 = key_ref[pl.ds(0, _G1)]
            lo, hi = jax.lax.fori_loop(
                0, _n, _search_body(k1), (lo_ref[...], hi_ref[...]))
            lo_ref[...] = lo
            hi_ref[...] = hi

    @pl.when(r == _R - 1)
    def _phase_b():
        k2 = key_ref[pl.ds(_G1, _R - _G1)]
        lo0 = jnp.zeros((_R - _G1, 1, 1), jnp.int32)
        hi0 = jnp.full((_R - _G1, 1, 1), _INT_MAX, jnp.int32)
        tau2, _ = jax.lax.fori_loop(0, 32, _search_body(k2), (lo0, hi0))

        key = key_ref[...]
        dd = d_ref[...]
        tau = jnp.concatenate([lo_ref[...], tau2], axis=0)

        mask_ge = key >= tau
        count_ge = jnp.sum(mask_ge.astype(jnp.int32), axis=(1, 2),
                           keepdims=True)
        sum_ge = jnp.sum(jnp.where(mask_ge, dd, 0.0), axis=(1, 2),
                         keepdims=True)
        exact = jnp.all(count_ge == _K)

        @pl.when(exact)
        def _no_ties():
            o_ref[0, 0] = jnp.sum(sum_ge) * (1.0 / _DENOM)

        @pl.when(jnp.logical_not(exact))
        def _ties():
            mask_eq = key == tau
            count_eq = jnp.sum(mask_eq.astype(jnp.float32), axis=(1, 2),
                               keepdims=True)
            sum_eq = jnp.sum(jnp.where(mask_eq, dd, 0.0), axis=(1, 2),
                             keepdims=True)
            count_gt = count_ge.astype(jnp.float32) - count_eq
            sum_gt = sum_ge - sum_eq
            need = jnp.float32(_K) - count_gt
            total = jnp.sum(sum_gt + need * sum_eq / count_eq)
            o_ref[0, 0] = total * (1.0 / _DENOM)


def kernel(out_preds, out_targets, tl, tv, x_rep, in_x, in_l, in_v, in_n):
    t = out_targets.reshape(_R, _C, _H, _W)
    p = out_preds.reshape(_R, _C, _H, _W)
    g = jnp.asarray(_E)
    out = pl.pallas_call(
        _mae_body,
        grid=(_R,),
        in_specs=[
            pl.BlockSpec((1, _C, _H, _W), lambda r: (r, 0, 0, 0)),
            pl.BlockSpec((1, _C, _H, _W), lambda r: (r, 0, 0, 0)),
            pl.BlockSpec((1, _H, _W), lambda r: (r, 0, 0)),
        ],
        out_specs=pl.BlockSpec((1, 1), lambda r: (0, 0), memory_space=pltpu.SMEM),
        out_shape=jax.ShapeDtypeStruct((1, 1), jnp.float32),
        scratch_shapes=[
            pltpu.VMEM((_R, _H, _W), jnp.int32),
            pltpu.VMEM((_R, _H, _W), jnp.float32),
            pltpu.VMEM((_G1, 1, 1), jnp.int32),
            pltpu.VMEM((_G1, 1, 1), jnp.int32),
        ],
    )(t, p, g)
    return out[0, 0]
